# baseline (speedup 1.0000x reference)
import functools

import jax
import jax.numpy as jnp
from jax import lax
from jax.experimental import pallas as pl
from jax.experimental.pallas import tpu as pltpu
from jax.experimental.pallas import tpu_sc as plsc

_SCALE = 12.0
_NUM_CORES = 2
_NUM_SUBCORES = 16
_NW = _NUM_CORES * _NUM_SUBCORES
_LANES = 16
_CHUNK = 128


def _body(n_rows, d, ids_hbm, table_hbm, out_hbm, idx_v, rows_v, sem):
    per_w = n_rows // _NW
    wid = lax.axis_index("s") * _NUM_CORES + lax.axis_index("c")
    base = wid * per_w
    pltpu.sync_copy(ids_hbm.at[pl.ds(base, per_w)], idx_v)
    n_chunks = per_w // _CHUNK

    def chunk_body(c, carry):
        row0 = c * _CHUNK
        pltpu.async_copy(
            table_hbm.at[idx_v.at[pl.ds(row0, _CHUNK)]], rows_v, sem
        ).wait()

        def scale_row(r, carry2):
            for j in range(d // _LANES):
                sl = pl.ds(j * _LANES, _LANES)
                rows_v[r, sl] = rows_v[r, sl] * _SCALE
            return carry2

        lax.fori_loop(0, _CHUNK, scale_row, 0, unroll=4)
        pltpu.sync_copy(rows_v, out_hbm.at[pl.ds(base + row0, _CHUNK)])
        return carry

    lax.fori_loop(0, n_chunks, chunk_body, 0)


@functools.partial(jax.jit, static_argnames=())
def kernel(input_ids, weight):
    b, s = input_ids.shape
    v, d = weight.shape
    n_rows = b * s
    flat_ids = input_ids.reshape(n_rows)

    mesh = plsc.VectorSubcoreMesh(core_axis_name="c", subcore_axis_name="s")
    per_w = n_rows // _NW
    run = functools.partial(
        pl.kernel,
        mesh=mesh,
        out_type=jax.ShapeDtypeStruct((n_rows, d), jnp.float32),
        scratch_types=[
            pltpu.VMEM((per_w,), jnp.int32),
            pltpu.VMEM((_CHUNK, d), jnp.float32),
            pltpu.SemaphoreType.DMA,
        ],
        compiler_params=pltpu.CompilerParams(use_tc_tiling_on_sc=False),
    )(functools.partial(_body, n_rows, d))
    out = run(flat_ids, weight)
    return out.reshape(b, s, d)


# speedup vs baseline: 1.0567x; 1.0567x over previous
import functools

import jax
import jax.numpy as jnp
from jax import lax
from jax.experimental import pallas as pl
from jax.experimental.pallas import tpu as pltpu
from jax.experimental.pallas import tpu_sc as plsc

_SCALE = 12.0
_NUM_CORES = 2
_NUM_SUBCORES = 16
_NW = _NUM_CORES * _NUM_SUBCORES
_LANES = 16
_CHUNK = 128
_NBUF = 4


def _body(n_rows, d, ids_hbm, table_hbm, out_hbm, idx_v, in_v, out_v, gsem, ssem):
    per_w = n_rows // _NW
    wid = lax.axis_index("s") * _NUM_CORES + lax.axis_index("c")
    base = wid * per_w
    pltpu.sync_copy(ids_hbm.at[pl.ds(base, per_w)], idx_v)
    n_chunks = per_w // _CHUNK

    def gather(g, b):
        return pltpu.make_async_copy(
            table_hbm.at[idx_v.at[pl.ds(g * _CHUNK, _CHUNK)]],
            in_v.at[b],
            gsem.at[b],
        )

    def store(g, b):
        return pltpu.make_async_copy(
            out_v.at[b],
            out_hbm.at[pl.ds(base + g * _CHUNK, _CHUNK)],
            ssem.at[b],
        )

    for b in range(_NBUF):
        gather(b, b).start()

    def outer_body(o, carry):
        for b in range(_NBUF):
            g = o * _NBUF + b
            gather(g, b).wait()

            @pl.when(g >= _NBUF)
            def _():
                store(g - _NBUF, b).wait()

            def scale_rows(r, c2):
                for u in range(4):
                    for j in range(d // _LANES):
                        sl = pl.ds(j * _LANES, _LANES)
                        out_v[b, r * 4 + u, sl] = in_v[b, r * 4 + u, sl] * _SCALE
                return c2

            lax.fori_loop(0, _CHUNK // 4, scale_rows, 0, unroll=2)

            @pl.when(g + _NBUF < n_chunks)
            def _():
                gather(g + _NBUF, b).start()

            store(g, b).start()
        return carry

    lax.fori_loop(0, n_chunks // _NBUF, outer_body, 0)

    for b in range(_NBUF):
        store(n_chunks - _NBUF + b, b).wait()


@jax.jit
def kernel(input_ids, weight):
    b, s = input_ids.shape
    v, d = weight.shape
    n_rows = b * s
    flat_ids = input_ids.reshape(n_rows)

    mesh = plsc.VectorSubcoreMesh(core_axis_name="c", subcore_axis_name="s")
    per_w = n_rows // _NW
    run = functools.partial(
        pl.kernel,
        mesh=mesh,
        out_type=jax.ShapeDtypeStruct((n_rows, d), jnp.float32),
        scratch_types=[
            pltpu.VMEM((per_w,), jnp.int32),
            pltpu.VMEM((_NBUF, _CHUNK, d), jnp.float32),
            pltpu.VMEM((_NBUF, _CHUNK, d), jnp.float32),
            pltpu.SemaphoreType.DMA((_NBUF,)),
            pltpu.SemaphoreType.DMA((_NBUF,)),
        ],
        compiler_params=pltpu.CompilerParams(use_tc_tiling_on_sc=False),
    )(functools.partial(_body, n_rows, d))
    out = run(flat_ids, weight)
    return out.reshape(b, s, d)
